# baseline (speedup 1.0000x reference)
import functools

import jax
import jax.numpy as jnp
from jax import lax
from jax.experimental import pallas as pl
from jax.experimental.pallas import tpu as pltpu
from jax.experimental.pallas import tpu_sc as plsc


def _table_body(hd, xd_ref, w1_ref, b1_ref, wh_ref, bh_ref, skip_ref,
                g2_ref, beta_ref):
    t = lax.dot_general(xd_ref[...], w1_ref[...], (((1,), (1,)), ((), ())),
                        preferred_element_type=jnp.float32)
    t = jnp.maximum(t + b1_ref[...], 0.0)
    p = lax.dot_general(t, wh_ref[...], (((1,), (1,)), ((), ())),
                        preferred_element_type=jnp.float32)
    p = p + bh_ref[...]
    gamma = p[:, :hd]
    beta = p[:, hd:]
    g2 = 1.0 + jnp.tanh(gamma)
    s = skip_ref[0, 0]
    g2_ref[...] = g2 * (1.0 - s) + s
    beta_ref[...] = beta * (1.0 - s)


def _make_tables(X_d, W1, b1, Wh, bh, skipf):
    B, _ = X_d.shape
    hd = Wh.shape[0] // 2
    return pl.pallas_call(
        functools.partial(_table_body, hd),
        out_shape=(jax.ShapeDtypeStruct((B, hd), jnp.float32),
                   jax.ShapeDtypeStruct((B, hd), jnp.float32)),
    )(X_d, W1, b1.reshape(1, -1), Wh, bh.reshape(1, -1), skipf)


def _film_sc(H, ids, g2, beta):
    N, D = H.shape
    C = 128
    n_full = N // C
    tail = N - n_full * C
    info = plsc.get_sparse_core_info()
    nc, ns = info.num_cores, info.num_subcores
    nw = nc * ns
    rounds = (n_full + nw - 1) // nw
    mesh = plsc.VectorSubcoreMesh(core_axis_name="c", subcore_axis_name="s")

    @functools.partial(
        pl.kernel,
        mesh=mesh,
        out_type=jax.ShapeDtypeStruct((N, D), jnp.float32),
        scratch_types=[
            pltpu.VMEM((C,), jnp.int32),
            pltpu.VMEM((C, D), jnp.float32),
            pltpu.VMEM((C, D), jnp.float32),
            pltpu.VMEM((C, D), jnp.float32),
            pltpu.SemaphoreType.DMA,
        ],
    )
    def k(h_hbm, ids_hbm, g2_hbm, beta_hbm, out_hbm, ids_v, h_v, g_v, o_v, sem):
        wid = lax.axis_index("s") * nc + lax.axis_index("c")

        def do_rows(nrows):
            def row(i, carry):
                for kk in range(D // 16):
                    sl = pl.ds(kk * 16, 16)
                    plsc.addupdate(o_v.at[i, sl], g_v[i, sl] * h_v[i, sl])
                return carry
            lax.fori_loop(0, nrows, row, None)

        def do_chunk(c):
            base = c * C
            pltpu.sync_copy(ids_hbm.at[pl.ds(base, C)], ids_v)
            cp_g = pltpu.async_copy(g2_hbm.at[ids_v], g_v, sem)
            cp_b = pltpu.async_copy(beta_hbm.at[ids_v], o_v, sem)
            cp_h = pltpu.async_copy(h_hbm.at[pl.ds(base, C)], h_v, sem)
            cp_g.wait()
            cp_b.wait()
            cp_h.wait()
            do_rows(C)
            pltpu.sync_copy(o_v, out_hbm.at[pl.ds(base, C)])

        def chunk_loop(j, carry):
            c = wid + j * nw

            @pl.when(c < n_full)
            def _():
                do_chunk(c)
            return carry

        lax.fori_loop(0, rounds, chunk_loop, None)

        if tail:
            @pl.when(wid == nw - 1)
            def _():
                base = n_full * C
                pltpu.sync_copy(ids_hbm.at[pl.ds(base, tail)],
                                ids_v.at[pl.ds(0, tail)])
                cp_g = pltpu.async_copy(g2_hbm.at[ids_v.at[pl.ds(0, tail)]],
                                        g_v.at[pl.ds(0, tail)], sem)
                cp_b = pltpu.async_copy(beta_hbm.at[ids_v.at[pl.ds(0, tail)]],
                                        o_v.at[pl.ds(0, tail)], sem)
                cp_h = pltpu.async_copy(h_hbm.at[pl.ds(base, tail)],
                                        h_v.at[pl.ds(0, tail)], sem)
                cp_g.wait()
                cp_b.wait()
                cp_h.wait()
                do_rows(tail)
                pltpu.sync_copy(o_v.at[pl.ds(0, tail)],
                                out_hbm.at[pl.ds(base, tail)])

    return k(H, ids, g2, beta)


def kernel(H, X_d, layer_idx, graph_ids, W1, b1, Wh, bh):
    skipf = (jnp.asarray(layer_idx) >= 4).astype(jnp.float32).reshape(1, 1)
    g2, beta = _make_tables(X_d, W1, b1, Wh, bh, skipf)
    ids = graph_ids.astype(jnp.int32)
    return _film_sc(H, ids, g2, beta)


# speedup vs baseline: 1.3527x; 1.3527x over previous
import functools

import jax
import jax.numpy as jnp
from jax import lax
from jax.experimental import pallas as pl
from jax.experimental.pallas import tpu as pltpu
from jax.experimental.pallas import tpu_sc as plsc


def _table_body(hd, xd_ref, w1_ref, b1_ref, wh_ref, bh_ref, skip_ref,
                g2_ref, beta_ref):
    t = lax.dot_general(xd_ref[...], w1_ref[...], (((1,), (1,)), ((), ())),
                        preferred_element_type=jnp.float32)
    t = jnp.maximum(t + b1_ref[...], 0.0)
    p = lax.dot_general(t, wh_ref[...], (((1,), (1,)), ((), ())),
                        preferred_element_type=jnp.float32)
    p = p + bh_ref[...]
    gamma = p[:, :hd]
    beta = p[:, hd:]
    g2 = 1.0 + jnp.tanh(gamma)
    s = skip_ref[0, 0]
    g2_ref[...] = g2 * (1.0 - s) + s
    beta_ref[...] = beta * (1.0 - s)


def _make_tables(X_d, W1, b1, Wh, bh, skipf):
    B, _ = X_d.shape
    hd = Wh.shape[0] // 2
    return pl.pallas_call(
        functools.partial(_table_body, hd),
        out_shape=(jax.ShapeDtypeStruct((B, hd), jnp.float32),
                   jax.ShapeDtypeStruct((B, hd), jnp.float32)),
    )(X_d, W1, b1.reshape(1, -1), Wh, bh.reshape(1, -1), skipf)


def _film_sc(H, ids, g2, beta):
    N, D = H.shape
    C = 128
    n_full = N // C
    tail = N - n_full * C
    info = plsc.get_sparse_core_info()
    nc, ns = info.num_cores, info.num_subcores
    nw = nc * ns
    rmax = (n_full + nw - 1) // nw
    tmax = (rmax + 1) // 2
    mesh = plsc.VectorSubcoreMesh(core_axis_name="c", subcore_axis_name="s")

    @functools.partial(
        pl.kernel,
        mesh=mesh,
        out_type=jax.ShapeDtypeStruct((N, D), jnp.float32),
        scratch_types=[
            pltpu.VMEM((C,), jnp.int32),
            pltpu.VMEM((C,), jnp.int32),
            pltpu.VMEM((C, D), jnp.float32),
            pltpu.VMEM((C, D), jnp.float32),
            pltpu.VMEM((C, D), jnp.float32),
            pltpu.VMEM((C, D), jnp.float32),
            pltpu.VMEM((C, D), jnp.float32),
            pltpu.VMEM((C, D), jnp.float32),
            pltpu.SemaphoreType.DMA,
            pltpu.SemaphoreType.DMA,
            pltpu.SemaphoreType.DMA,
            pltpu.SemaphoreType.DMA,
        ],
    )
    def k(h_hbm, ids_hbm, g2_hbm, beta_hbm, out_hbm,
          i0, i1, h0, h1, g0, g1, o0, o1, s0, s1, si0, si1):
        wid = lax.axis_index("s") * nc + lax.axis_index("c")
        ib = (i0, i1)
        hb = (h0, h1)
        gb = (g0, g1)
        ob = (o0, o1)
        sem = (s0, s1)
        semi = (si0, si1)

        def start_ids(c, b):
            pltpu.async_copy(ids_hbm.at[pl.ds(c * C, C)], ib[b], semi[b])

        def wait_ids(b):
            pltpu.make_async_copy(ids_hbm.at[pl.ds(0, C)],
                                  ib[b], semi[b]).wait()

        def start_gathers(c, b):
            pltpu.async_copy(g2_hbm.at[ib[b]], gb[b], sem[b])
            pltpu.async_copy(beta_hbm.at[ib[b]], ob[b], sem[b])
            pltpu.async_copy(h_hbm.at[pl.ds(c * C, C)], hb[b], sem[b])

        def wait_gathers(b):
            pltpu.make_async_copy(g2_hbm.at[ib[b]], gb[b], sem[b]).wait()
            pltpu.make_async_copy(beta_hbm.at[ib[b]], ob[b], sem[b]).wait()
            pltpu.make_async_copy(h_hbm.at[pl.ds(0, C)], hb[b], sem[b]).wait()

        def do_rows(b, nrows):
            g_v, h_v, o_v = gb[b], hb[b], ob[b]

            def row(i, carry):
                for kk in range(D // 16):
                    sl = pl.ds(kk * 16, 16)
                    plsc.addupdate(o_v.at[i, sl], g_v[i, sl] * h_v[i, sl])
                return carry
            lax.fori_loop(0, nrows, row, None)

        c0 = wid

        @pl.when(c0 < n_full)
        def _():
            start_ids(c0, 0)
            wait_ids(0)
            start_gathers(c0, 0)

        @pl.when(c0 + nw < n_full)
        def _():
            start_ids(c0 + nw, 1)

        def pair(t, carry):
            ca = wid + (2 * t) * nw
            cb_ = ca + nw

            @pl.when(cb_ < n_full)
            def _():
                wait_ids(1)
                start_gathers(cb_, 1)

            @pl.when(ca < n_full)
            def _():
                wait_gathers(0)

            @pl.when(ca + 2 * nw < n_full)
            def _():
                start_ids(ca + 2 * nw, 0)

            @pl.when(ca < n_full)
            def _():
                do_rows(0, C)
                pltpu.sync_copy(o0, out_hbm.at[pl.ds(ca * C, C)])

            @pl.when(ca + 2 * nw < n_full)
            def _():
                wait_ids(0)
                start_gathers(ca + 2 * nw, 0)

            @pl.when(cb_ < n_full)
            def _():
                wait_gathers(1)

            @pl.when(cb_ + 2 * nw < n_full)
            def _():
                start_ids(cb_ + 2 * nw, 1)

            @pl.when(cb_ < n_full)
            def _():
                do_rows(1, C)
                pltpu.sync_copy(o1, out_hbm.at[pl.ds(cb_ * C, C)])
            return carry

        lax.fori_loop(0, tmax, pair, None)

        if tail:
            @pl.when(wid == nw - 1)
            def _():
                pltpu.sync_copy(ids_hbm.at[pl.ds(n_full * C, tail)],
                                i0.at[pl.ds(0, tail)])
                idx = i0.at[pl.ds(0, tail)]
                cg = pltpu.async_copy(g2_hbm.at[idx],
                                      g0.at[pl.ds(0, tail)], s0)
                cb = pltpu.async_copy(beta_hbm.at[idx],
                                      o0.at[pl.ds(0, tail)], s0)
                ch = pltpu.async_copy(h_hbm.at[pl.ds(n_full * C, tail)],
                                      h0.at[pl.ds(0, tail)], s0)
                cg.wait()
                cb.wait()
                ch.wait()
                do_rows(0, tail)
                pltpu.sync_copy(o0.at[pl.ds(0, tail)],
                                out_hbm.at[pl.ds(n_full * C, tail)])

    return k(H, ids, g2, beta)


def kernel(H, X_d, layer_idx, graph_ids, W1, b1, Wh, bh):
    skipf = (jnp.asarray(layer_idx) >= 4).astype(jnp.float32).reshape(1, 1)
    g2, beta = _make_tables(X_d, W1, b1, Wh, bh, skipf)
    ids = graph_ids.astype(jnp.int32)
    return _film_sc(H, ids, g2, beta)


# speedup vs baseline: 1.6392x; 1.2118x over previous
import functools

import jax
import jax.numpy as jnp
from jax import lax
from jax.experimental import pallas as pl
from jax.experimental.pallas import tpu as pltpu
from jax.experimental.pallas import tpu_sc as plsc


def _table_body(hd, xd_ref, w1_ref, b1_ref, wh_ref, bh_ref, skip_ref,
                g2_ref, beta_ref):
    t = lax.dot_general(xd_ref[...], w1_ref[...], (((1,), (1,)), ((), ())),
                        preferred_element_type=jnp.float32)
    t = jnp.maximum(t + b1_ref[...], 0.0)
    p = lax.dot_general(t, wh_ref[...], (((1,), (1,)), ((), ())),
                        preferred_element_type=jnp.float32)
    p = p + bh_ref[...]
    gamma = p[:, :hd]
    beta = p[:, hd:]
    g2 = 1.0 + jnp.tanh(gamma)
    s = skip_ref[0, 0]
    g2_ref[...] = g2 * (1.0 - s) + s
    beta_ref[...] = beta * (1.0 - s)


def _make_tables(X_d, W1, b1, Wh, bh, skipf):
    B, _ = X_d.shape
    hd = Wh.shape[0] // 2
    return pl.pallas_call(
        functools.partial(_table_body, hd),
        out_shape=(jax.ShapeDtypeStruct((B, hd), jnp.float32),
                   jax.ShapeDtypeStruct((B, hd), jnp.float32)),
    )(X_d, W1, b1.reshape(1, -1), Wh, bh.reshape(1, -1), skipf)


def _film_sc(H, ids, g2f, betaf, rs_pad):
    N, D = H.shape
    NV = D // 16
    C = 128
    n_full = N // C
    tail = N - n_full * C
    tabw = g2f.shape[0]
    rsw = rs_pad.shape[0]
    info = plsc.get_sparse_core_info()
    nc, ns = info.num_cores, info.num_subcores
    nw = nc * ns
    rmax = (n_full + nw - 1) // nw
    tmax = (rmax + 1) // 2
    mesh = plsc.VectorSubcoreMesh(core_axis_name="c", subcore_axis_name="s")

    @functools.partial(
        pl.kernel,
        mesh=mesh,
        out_type=jax.ShapeDtypeStruct((N, D), jnp.float32),
        scratch_types=[
            pltpu.VMEM((C + 16,), jnp.int32),
            pltpu.VMEM((C + 16,), jnp.int32),
            pltpu.VMEM((C, D), jnp.float32),
            pltpu.VMEM((C, D), jnp.float32),
            pltpu.VMEM((C, D), jnp.float32),
            pltpu.VMEM((C, D), jnp.float32),
            pltpu.VMEM((rsw,), jnp.int32),
            pltpu.VMEM((D,), jnp.float32),
            pltpu.VMEM((D,), jnp.float32),
            pltpu.VMEM_SHARED((tabw,), jnp.float32),
            pltpu.VMEM_SHARED((tabw,), jnp.float32),
            pltpu.SemaphoreType.DMA,
            pltpu.SemaphoreType.DMA,
            pltpu.SemaphoreType.DMA,
            pltpu.SemaphoreType.DMA,
        ],
    )
    def k(h_hbm, ids_hbm, g2_hbm, beta_hbm, rs_hbm, out_hbm,
          i0, i1, h0, h1, o0, o1, rs_v, gp, bp, spg, spb,
          s0, s1, so0, so1):
        wid = lax.axis_index("s") * nc + lax.axis_index("c")
        ib = (i0, i1)
        hb = (h0, h1)
        ob = (o0, o1)
        sem = (s0, s1)
        semo = (so0, so1)

        @pl.when(lax.axis_index("s") == 0)
        def _():
            pltpu.sync_copy(g2_hbm, spg)
            pltpu.sync_copy(beta_hbm, spb)
        plsc.subcore_barrier()
        pltpu.sync_copy(rs_hbm, rs_v)

        def extract(ref, idx):
            return ref[pl.ds(idx, 16)][0]

        def start_in(c, b):
            pltpu.async_copy(ids_hbm.at[pl.ds(c * C, C)],
                             ib[b].at[pl.ds(0, C)], sem[b])
            pltpu.async_copy(h_hbm.at[pl.ds(c * C, C)], hb[b], sem[b])

        def wait_in(b):
            pltpu.make_async_copy(ids_hbm.at[pl.ds(0, C)],
                                  ib[b].at[pl.ds(0, C)], sem[b]).wait()
            pltpu.make_async_copy(h_hbm.at[pl.ds(0, C)],
                                  hb[b], sem[b]).wait()

        def start_store(c, b):
            pltpu.async_copy(ob[b], out_hbm.at[pl.ds(c * C, C)], semo[b])

        def wait_store(b):
            pltpu.make_async_copy(ob[b], out_hbm.at[pl.ds(0, C)],
                                  semo[b]).wait()

        def seg_walk(base, nrows, b):
            ids_v, h_v, o_v = ib[b], hb[b], ob[b]
            q0 = extract(ids_v, 0)
            q1 = extract(ids_v, nrows - 1)

            def seg(qi, r):
                r_end = jnp.minimum(extract(rs_v, qi + 1) - base, nrows)
                pltpu.sync_copy(spg.at[pl.ds(qi * D, D)], gp)
                pltpu.sync_copy(spb.at[pl.ds(qi * D, D)], bp)
                gr = [gp[pl.ds(kk * 16, 16)] for kk in range(NV)]
                br = [bp[pl.ds(kk * 16, 16)] for kk in range(NV)]

                def row(i, carry):
                    for kk in range(NV):
                        sl = pl.ds(kk * 16, 16)
                        o_v[i, sl] = gr[kk] * h_v[i, sl] + br[kk]
                    return carry
                lax.fori_loop(r, r_end, row, None)
                return r_end

            lax.fori_loop(q0, q1 + 1, seg, jnp.int32(0))

        c0 = wid

        @pl.when(c0 < n_full)
        def _():
            start_in(c0, 0)

        @pl.when(c0 + nw < n_full)
        def _():
            start_in(c0 + nw, 1)

        def pair(t, carry):
            ca = wid + (2 * t) * nw
            cb_ = ca + nw

            @pl.when(ca < n_full)
            def _():
                wait_in(0)

                @pl.when(t >= 1)
                def _():
                    wait_store(0)
                seg_walk(ca * C, C, 0)
                start_store(ca, 0)

            @pl.when(ca + 2 * nw < n_full)
            def _():
                start_in(ca + 2 * nw, 0)

            @pl.when(cb_ < n_full)
            def _():
                wait_in(1)

                @pl.when(t >= 1)
                def _():
                    wait_store(1)
                seg_walk(cb_ * C, C, 1)
                start_store(cb_, 1)

            @pl.when(cb_ + 2 * nw < n_full)
            def _():
                start_in(cb_ + 2 * nw, 1)
            return carry

        lax.fori_loop(0, tmax, pair, None)

        ce0 = wid + (2 * (tmax - 1)) * nw

        @pl.when(ce0 < n_full)
        def _():
            wait_store(0)

        @pl.when(ce0 + nw < n_full)
        def _():
            wait_store(1)

        if tail:
            @pl.when(wid == nw - 1)
            def _():
                pltpu.sync_copy(ids_hbm.at[pl.ds(n_full * C, tail)],
                                i0.at[pl.ds(0, tail)])
                pltpu.sync_copy(h_hbm.at[pl.ds(n_full * C, tail)],
                                h0.at[pl.ds(0, tail)])
                seg_walk(n_full * C, tail, 0)
                pltpu.sync_copy(o0.at[pl.ds(0, tail)],
                                out_hbm.at[pl.ds(n_full * C, tail)])

    return k(H, ids, g2f, betaf, rs_pad)


def kernel(H, X_d, layer_idx, graph_ids, W1, b1, Wh, bh):
    skipf = (jnp.asarray(layer_idx) >= 4).astype(jnp.float32).reshape(1, 1)
    g2, beta = _make_tables(X_d, W1, b1, Wh, bh, skipf)
    ids = graph_ids.astype(jnp.int32)
    B = X_d.shape[0]
    rs = jnp.searchsorted(ids, jnp.arange(B + 1, dtype=jnp.int32),
                          side="left").astype(jnp.int32)
    rsw = ((B + 1 + 15) // 16 + 1) * 16
    rs_pad = jnp.zeros((rsw,), jnp.int32).at[:B + 1].set(rs)
    return _film_sc(H, ids, g2.reshape(-1), beta.reshape(-1), rs_pad)


# speedup vs baseline: 2.1538x; 1.3139x over previous
import functools

import jax
import jax.numpy as jnp
from jax import lax
from jax.experimental import pallas as pl
from jax.experimental.pallas import tpu as pltpu
from jax.experimental.pallas import tpu_sc as plsc


def _table_body(hd, xd_ref, w1_ref, b1_ref, wh_ref, bh_ref, skip_ref,
                g2_ref, beta_ref):
    t = lax.dot_general(xd_ref[...], w1_ref[...], (((1,), (1,)), ((), ())),
                        preferred_element_type=jnp.float32)
    t = jnp.maximum(t + b1_ref[...], 0.0)
    p = lax.dot_general(t, wh_ref[...], (((1,), (1,)), ((), ())),
                        preferred_element_type=jnp.float32)
    p = p + bh_ref[...]
    gamma = p[:, :hd]
    beta = p[:, hd:]
    g2 = 1.0 + jnp.tanh(gamma)
    s = skip_ref[0, 0]
    g2_ref[...] = g2 * (1.0 - s) + s
    beta_ref[...] = beta * (1.0 - s)


def _make_tables(X_d, W1, b1, Wh, bh, skipf):
    B, _ = X_d.shape
    hd = Wh.shape[0] // 2
    return pl.pallas_call(
        functools.partial(_table_body, hd),
        out_shape=(jax.ShapeDtypeStruct((B, hd), jnp.float32),
                   jax.ShapeDtypeStruct((B, hd), jnp.float32)),
    )(X_d, W1, b1.reshape(1, -1), Wh, bh.reshape(1, -1), skipf)


def _film_sc(H, ids, g2f, betaf, rs_pad):
    N, D = H.shape
    NV = D // 16
    C = 128
    n_full = N // C
    tail = N - n_full * C
    tabw = g2f.shape[0]
    rsw = rs_pad.shape[0]
    info = plsc.get_sparse_core_info()
    nc, ns = info.num_cores, info.num_subcores
    nw = nc * ns
    rmax = (n_full + nw - 1) // nw
    tmax = (rmax + 1) // 2
    mesh = plsc.VectorSubcoreMesh(core_axis_name="c", subcore_axis_name="s")

    @functools.partial(
        pl.kernel,
        mesh=mesh,
        out_type=jax.ShapeDtypeStruct((N, D), jnp.float32),
        scratch_types=[
            pltpu.VMEM((C + 16,), jnp.int32),
            pltpu.VMEM((C + 16,), jnp.int32),
            pltpu.VMEM((C, D), jnp.float32),
            pltpu.VMEM((C, D), jnp.float32),
            pltpu.VMEM((C, D), jnp.float32),
            pltpu.VMEM((C, D), jnp.float32),
            pltpu.VMEM((rsw,), jnp.int32),
            pltpu.VMEM((D,), jnp.float32),
            pltpu.VMEM((D,), jnp.float32),
            pltpu.VMEM_SHARED((tabw,), jnp.float32),
            pltpu.VMEM_SHARED((tabw,), jnp.float32),
            pltpu.SemaphoreType.DMA,
            pltpu.SemaphoreType.DMA,
            pltpu.SemaphoreType.DMA,
            pltpu.SemaphoreType.DMA,
        ],
    )
    def k(h_hbm, ids_hbm, g2_hbm, beta_hbm, rs_hbm, out_hbm,
          i0, i1, h0, h1, o0, o1, rs_v, gp, bp, spg, spb,
          s0, s1, so0, so1):
        wid = lax.axis_index("s") * nc + lax.axis_index("c")
        ib = (i0, i1)
        hb = (h0, h1)
        ob = (o0, o1)
        sem = (s0, s1)
        semo = (so0, so1)

        @pl.when(lax.axis_index("s") == 0)
        def _():
            pltpu.sync_copy(g2_hbm, spg)
            pltpu.sync_copy(beta_hbm, spb)
        plsc.subcore_barrier()
        pltpu.sync_copy(rs_hbm, rs_v)

        def extract(ref, idx):
            return ref[pl.ds(idx, 16)][0]

        def start_in(c, b):
            pltpu.async_copy(ids_hbm.at[pl.ds(c * C, C)],
                             ib[b].at[pl.ds(0, C)], sem[b])
            pltpu.async_copy(h_hbm.at[pl.ds(c * C, C)], hb[b], sem[b])

        def wait_in(b):
            pltpu.make_async_copy(ids_hbm.at[pl.ds(0, C)],
                                  ib[b].at[pl.ds(0, C)], sem[b]).wait()
            pltpu.make_async_copy(h_hbm.at[pl.ds(0, C)],
                                  hb[b], sem[b]).wait()

        def start_store(c, b):
            pltpu.async_copy(ob[b], out_hbm.at[pl.ds(c * C, C)], semo[b])

        def wait_store(b):
            pltpu.make_async_copy(ob[b], out_hbm.at[pl.ds(0, C)],
                                  semo[b]).wait()

        def seg_walk(base, nrows, b):
            ids_v, h_v, o_v = ib[b], hb[b], ob[b]
            q0 = extract(ids_v, 0)
            q1 = extract(ids_v, nrows - 1)

            def load_params(qi):
                pltpu.sync_copy(spg.at[pl.ds(qi * D, D)], gp)
                pltpu.sync_copy(spb.at[pl.ds(qi * D, D)], bp)
                gr = [gp[pl.ds(kk * 16, 16)] for kk in range(NV)]
                br = [bp[pl.ds(kk * 16, 16)] for kk in range(NV)]
                return gr, br

            def make_row(gr, br):
                def row(i, carry):
                    for kk in range(NV):
                        sl = pl.ds(kk * 16, 16)
                        o_v[i, sl] = gr[kk] * h_v[i, sl] + br[kk]
                    return carry
                return row

            gr0, br0 = load_params(q0)
            lax.fori_loop(0, nrows, make_row(gr0, br0), None)

            @pl.when(q1 != q0)
            def _():
                def seg(qi, r):
                    r_end = jnp.minimum(extract(rs_v, qi + 1) - base, nrows)
                    gr, br = load_params(qi)
                    lax.fori_loop(r, r_end, make_row(gr, br), None)
                    return r_end

                r1 = jnp.minimum(extract(rs_v, q0 + 1) - base, nrows)
                lax.fori_loop(q0 + 1, q1 + 1, seg, r1)

        c0 = wid

        @pl.when(c0 < n_full)
        def _():
            start_in(c0, 0)

        @pl.when(c0 + nw < n_full)
        def _():
            start_in(c0 + nw, 1)

        def pair(t, carry):
            ca = wid + (2 * t) * nw
            cb_ = ca + nw

            @pl.when(ca < n_full)
            def _():
                wait_in(0)

                @pl.when(t >= 1)
                def _():
                    wait_store(0)
                seg_walk(ca * C, C, 0)
                start_store(ca, 0)

            @pl.when(ca + 2 * nw < n_full)
            def _():
                start_in(ca + 2 * nw, 0)

            @pl.when(cb_ < n_full)
            def _():
                wait_in(1)

                @pl.when(t >= 1)
                def _():
                    wait_store(1)
                seg_walk(cb_ * C, C, 1)
                start_store(cb_, 1)

            @pl.when(cb_ + 2 * nw < n_full)
            def _():
                start_in(cb_ + 2 * nw, 1)
            return carry

        lax.fori_loop(0, tmax, pair, None)

        ce0 = wid + (2 * (tmax - 1)) * nw

        @pl.when(ce0 < n_full)
        def _():
            wait_store(0)

        @pl.when(ce0 + nw < n_full)
        def _():
            wait_store(1)

        if tail:
            @pl.when(wid == nw - 1)
            def _():
                pltpu.sync_copy(ids_hbm.at[pl.ds(n_full * C, tail)],
                                i0.at[pl.ds(0, tail)])
                pltpu.sync_copy(h_hbm.at[pl.ds(n_full * C, tail)],
                                h0.at[pl.ds(0, tail)])
                seg_walk(n_full * C, tail, 0)
                pltpu.sync_copy(o0.at[pl.ds(0, tail)],
                                out_hbm.at[pl.ds(n_full * C, tail)])

    return k(H, ids, g2f, betaf, rs_pad)


def kernel(H, X_d, layer_idx, graph_ids, W1, b1, Wh, bh):
    skipf = (jnp.asarray(layer_idx) >= 4).astype(jnp.float32).reshape(1, 1)
    g2, beta = _make_tables(X_d, W1, b1, Wh, bh, skipf)
    ids = graph_ids.astype(jnp.int32)
    B = X_d.shape[0]
    rs = jnp.searchsorted(ids, jnp.arange(B + 1, dtype=jnp.int32),
                          side="left").astype(jnp.int32)
    rsw = ((B + 1 + 15) // 16 + 1) * 16
    rs_pad = jnp.zeros((rsw,), jnp.int32).at[:B + 1].set(rs)
    return _film_sc(H, ids, g2.reshape(-1), beta.reshape(-1), rs_pad)


# speedup vs baseline: 2.6976x; 1.2525x over previous
import functools

import jax
import jax.numpy as jnp
from jax import lax
from jax.experimental import pallas as pl
from jax.experimental.pallas import tpu as pltpu
from jax.experimental.pallas import tpu_sc as plsc


def _table_body(hd, xd_ref, w1_ref, b1_ref, wh_ref, bh_ref, skip_ref,
                g2_ref, beta_ref):
    t = lax.dot_general(xd_ref[...], w1_ref[...], (((1,), (1,)), ((), ())),
                        preferred_element_type=jnp.float32)
    t = jnp.maximum(t + b1_ref[...], 0.0)
    p = lax.dot_general(t, wh_ref[...], (((1,), (1,)), ((), ())),
                        preferred_element_type=jnp.float32)
    p = p + bh_ref[...]
    gamma = p[:, :hd]
    beta = p[:, hd:]
    g2 = 1.0 + jnp.tanh(gamma)
    s = skip_ref[0, 0]
    g2_ref[...] = g2 * (1.0 - s) + s
    beta_ref[...] = beta * (1.0 - s)


def _make_tables(X_d, W1, b1, Wh, bh, skipf):
    B, _ = X_d.shape
    hd = Wh.shape[0] // 2
    return pl.pallas_call(
        functools.partial(_table_body, hd),
        out_shape=(jax.ShapeDtypeStruct((B, hd), jnp.float32),
                   jax.ShapeDtypeStruct((B, hd), jnp.float32)),
    )(X_d, W1, b1.reshape(1, -1), Wh, bh.reshape(1, -1), skipf)


def _film_sc(H, ids, g2f, betaf, rs_pad):
    N, D = H.shape
    NV = D // 16
    C = 128
    n_full = N // C
    tail = N - n_full * C
    tabw = g2f.shape[0]
    rsw = rs_pad.shape[0]
    info = plsc.get_sparse_core_info()
    nc, ns = info.num_cores, info.num_subcores
    nw = nc * ns
    rmax = (n_full + nw - 1) // nw
    tmax = (rmax + 1) // 2
    mesh = plsc.VectorSubcoreMesh(core_axis_name="c", subcore_axis_name="s")

    @functools.partial(
        pl.kernel,
        mesh=mesh,
        out_type=jax.ShapeDtypeStruct((N, D), jnp.float32),
        scratch_types=[
            pltpu.VMEM((C + 16,), jnp.int32),
            pltpu.VMEM((C + 16,), jnp.int32),
            pltpu.VMEM((C, D), jnp.float32),
            pltpu.VMEM((C, D), jnp.float32),
            pltpu.VMEM((C, D), jnp.float32),
            pltpu.VMEM((C, D), jnp.float32),
            pltpu.VMEM((rsw,), jnp.int32),
            pltpu.VMEM((D,), jnp.float32),
            pltpu.VMEM((D,), jnp.float32),
            pltpu.VMEM_SHARED((tabw,), jnp.float32),
            pltpu.VMEM_SHARED((tabw,), jnp.float32),
            pltpu.SemaphoreType.DMA,
            pltpu.SemaphoreType.DMA,
            pltpu.SemaphoreType.DMA,
            pltpu.SemaphoreType.DMA,
        ],
    )
    def k(h_hbm, ids_hbm, g2_hbm, beta_hbm, rs_hbm, out_hbm,
          i0, i1, h0, h1, o0, o1, rs_v, gp, bp, spg, spb,
          s0, s1, so0, so1):
        wid = lax.axis_index("s") * nc + lax.axis_index("c")
        ib = (i0, i1)
        hb = (h0, h1)
        ob = (o0, o1)
        sem = (s0, s1)
        semo = (so0, so1)

        @pl.when(lax.axis_index("s") == 0)
        def _():
            pltpu.sync_copy(g2_hbm, spg)
            pltpu.sync_copy(beta_hbm, spb)
        plsc.subcore_barrier()
        pltpu.sync_copy(rs_hbm, rs_v)

        def extract(ref, idx):
            return ref[pl.ds(idx, 16)][0]

        def start_in(c, b):
            pltpu.async_copy(ids_hbm.at[pl.ds(c * C, C)],
                             ib[b].at[pl.ds(0, C)], sem[b])
            pltpu.async_copy(h_hbm.at[pl.ds(c * C, C)], hb[b], sem[b])

        def wait_in(b):
            pltpu.make_async_copy(ids_hbm.at[pl.ds(0, C)],
                                  ib[b].at[pl.ds(0, C)], sem[b]).wait()
            pltpu.make_async_copy(h_hbm.at[pl.ds(0, C)],
                                  hb[b], sem[b]).wait()

        def start_store(c, b):
            pltpu.async_copy(ob[b], out_hbm.at[pl.ds(c * C, C)], semo[b])

        def wait_store(b):
            pltpu.make_async_copy(ob[b], out_hbm.at[pl.ds(0, C)],
                                  semo[b]).wait()

        def seg_walk(base, nrows, b):
            ids_v, h_v, o_v = ib[b], hb[b], ob[b]
            q0 = extract(ids_v, 0)
            q1 = extract(ids_v, nrows - 1)

            def load_params(qi):
                pltpu.sync_copy(spg.at[pl.ds(qi * D, D)], gp)
                pltpu.sync_copy(spb.at[pl.ds(qi * D, D)], bp)
                gr = [gp[pl.ds(kk * 16, 16)] for kk in range(NV)]
                br = [bp[pl.ds(kk * 16, 16)] for kk in range(NV)]
                return gr, br

            def make_row(gr, br):
                def row(i, carry):
                    for kk in range(NV):
                        sl = pl.ds(kk * 16, 16)
                        o_v[i, sl] = gr[kk] * h_v[i, sl] + br[kk]
                    return carry
                return row

            gr0, br0 = load_params(q0)
            lax.fori_loop(0, nrows, make_row(gr0, br0), None)

            @pl.when(q1 != q0)
            def _():
                def seg(qi, r):
                    r_end = jnp.minimum(extract(rs_v, qi + 1) - base, nrows)
                    gr, br = load_params(qi)

                    def row(i, carry):
                        m = jnp.minimum(jnp.minimum(i - r + 1, r_end - i),
                                        1).astype(jnp.float32)
                        m = jnp.maximum(m, 0.0)
                        for kk in range(NV):
                            sl = pl.ds(kk * 16, 16)
                            new_v = gr[kk] * h_v[i, sl] + br[kk]
                            old_v = o_v[i, sl]
                            o_v[i, sl] = old_v + m * (new_v - old_v)
                        return carry
                    lax.fori_loop(0, nrows, row, None)
                    return r_end

                r1 = jnp.minimum(extract(rs_v, q0 + 1) - base, nrows)
                lax.fori_loop(q0 + 1, q1 + 1, seg, r1)

        c0 = wid

        @pl.when(c0 < n_full)
        def _():
            start_in(c0, 0)

        @pl.when(c0 + nw < n_full)
        def _():
            start_in(c0 + nw, 1)

        def pair(t, carry):
            ca = wid + (2 * t) * nw
            cb_ = ca + nw

            @pl.when(ca < n_full)
            def _():
                wait_in(0)

                @pl.when(t >= 1)
                def _():
                    wait_store(0)
                seg_walk(ca * C, C, 0)
                start_store(ca, 0)

            @pl.when(ca + 2 * nw < n_full)
            def _():
                start_in(ca + 2 * nw, 0)

            @pl.when(cb_ < n_full)
            def _():
                wait_in(1)

                @pl.when(t >= 1)
                def _():
                    wait_store(1)
                seg_walk(cb_ * C, C, 1)
                start_store(cb_, 1)

            @pl.when(cb_ + 2 * nw < n_full)
            def _():
                start_in(cb_ + 2 * nw, 1)
            return carry

        lax.fori_loop(0, tmax, pair, None)

        ce0 = wid + (2 * (tmax - 1)) * nw

        @pl.when(ce0 < n_full)
        def _():
            wait_store(0)

        @pl.when(ce0 + nw < n_full)
        def _():
            wait_store(1)

        if tail:
            @pl.when(wid == nw - 1)
            def _():
                pltpu.sync_copy(ids_hbm.at[pl.ds(n_full * C, tail)],
                                i0.at[pl.ds(0, tail)])
                pltpu.sync_copy(h_hbm.at[pl.ds(n_full * C, tail)],
                                h0.at[pl.ds(0, tail)])
                seg_walk(n_full * C, tail, 0)
                pltpu.sync_copy(o0.at[pl.ds(0, tail)],
                                out_hbm.at[pl.ds(n_full * C, tail)])

    return k(H, ids, g2f, betaf, rs_pad)


def kernel(H, X_d, layer_idx, graph_ids, W1, b1, Wh, bh):
    skipf = (jnp.asarray(layer_idx) >= 4).astype(jnp.float32).reshape(1, 1)
    g2, beta = _make_tables(X_d, W1, b1, Wh, bh, skipf)
    ids = graph_ids.astype(jnp.int32)
    B = X_d.shape[0]
    rs = jnp.searchsorted(ids, jnp.arange(B + 1, dtype=jnp.int32),
                          side="left").astype(jnp.int32)
    rsw = ((B + 1 + 15) // 16 + 1) * 16
    rs_pad = jnp.zeros((rsw,), jnp.int32).at[:B + 1].set(rs)
    return _film_sc(H, ids, g2.reshape(-1), beta.reshape(-1), rs_pad)


# speedup vs baseline: 2.7328x; 1.0131x over previous
import functools

import jax
import jax.numpy as jnp
from jax import lax
from jax.experimental import pallas as pl
from jax.experimental.pallas import tpu as pltpu
from jax.experimental.pallas import tpu_sc as plsc


def _table_body(hd, xd_ref, w1_ref, b1_ref, wh_ref, bh_ref, skip_ref,
                g2_ref, beta_ref):
    t = lax.dot_general(xd_ref[...], w1_ref[...], (((1,), (1,)), ((), ())),
                        preferred_element_type=jnp.float32)
    t = jnp.maximum(t + b1_ref[...], 0.0)
    p = lax.dot_general(t, wh_ref[...], (((1,), (1,)), ((), ())),
                        preferred_element_type=jnp.float32)
    p = p + bh_ref[...]
    gamma = p[:, :hd]
    beta = p[:, hd:]
    g2 = 1.0 + jnp.tanh(gamma)
    s = skip_ref[0, 0]
    g2_ref[...] = g2 * (1.0 - s) + s
    beta_ref[...] = beta * (1.0 - s)


def _make_tables(X_d, W1, b1, Wh, bh, skipf):
    B, _ = X_d.shape
    hd = Wh.shape[0] // 2
    return pl.pallas_call(
        functools.partial(_table_body, hd),
        out_shape=(jax.ShapeDtypeStruct((B, hd), jnp.float32),
                   jax.ShapeDtypeStruct((B, hd), jnp.float32)),
    )(X_d, W1, b1.reshape(1, -1), Wh, bh.reshape(1, -1), skipf)


def _film_sc(H, ids, g2f, betaf, rs_pad):
    N, D = H.shape
    NV = D // 16
    C = 128
    n_full = N // C
    tail = N - n_full * C
    tabw = g2f.shape[0]
    rsw = rs_pad.shape[0]
    info = plsc.get_sparse_core_info()
    nc, ns = info.num_cores, info.num_subcores
    nw = nc * ns
    rmax = (n_full + nw - 1) // nw
    tmax = (rmax + 1) // 2
    mesh = plsc.VectorSubcoreMesh(core_axis_name="c", subcore_axis_name="s")

    @functools.partial(
        pl.kernel,
        mesh=mesh,
        out_type=jax.ShapeDtypeStruct((N, D), jnp.float32),
        scratch_types=[
            pltpu.VMEM((C + 16,), jnp.int32),
            pltpu.VMEM((C + 16,), jnp.int32),
            pltpu.VMEM((C, D), jnp.float32),
            pltpu.VMEM((C, D), jnp.float32),
            pltpu.VMEM((C, D), jnp.float32),
            pltpu.VMEM((C, D), jnp.float32),
            pltpu.VMEM((rsw,), jnp.int32),
            pltpu.VMEM((D,), jnp.float32),
            pltpu.VMEM((D,), jnp.float32),
            pltpu.VMEM_SHARED((tabw,), jnp.float32),
            pltpu.VMEM_SHARED((tabw,), jnp.float32),
            pltpu.SemaphoreType.DMA,
            pltpu.SemaphoreType.DMA,
            pltpu.SemaphoreType.DMA,
            pltpu.SemaphoreType.DMA,
        ],
    )
    def k(h_hbm, ids_hbm, g2_hbm, beta_hbm, rs_hbm, out_hbm,
          i0, i1, h0, h1, o0, o1, rs_v, gp, bp, spg, spb,
          s0, s1, so0, so1):
        wid = lax.axis_index("s") * nc + lax.axis_index("c")
        ib = (i0, i1)
        hb = (h0, h1)
        ob = (o0, o1)
        sem = (s0, s1)
        semo = (so0, so1)

        @pl.when(lax.axis_index("s") == 0)
        def _():
            pltpu.sync_copy(g2_hbm, spg)
            pltpu.sync_copy(beta_hbm, spb)
        plsc.subcore_barrier()
        pltpu.sync_copy(rs_hbm, rs_v)

        def extract(ref, idx):
            return ref[pl.ds(idx, 16)][0]

        def start_in(c, b):
            pltpu.async_copy(ids_hbm.at[pl.ds(c * C, C)],
                             ib[b].at[pl.ds(0, C)], sem[b])
            pltpu.async_copy(h_hbm.at[pl.ds(c * C, C)], hb[b], sem[b])

        def wait_in(b):
            pltpu.make_async_copy(ids_hbm.at[pl.ds(0, C)],
                                  ib[b].at[pl.ds(0, C)], sem[b]).wait()
            pltpu.make_async_copy(h_hbm.at[pl.ds(0, C)],
                                  hb[b], sem[b]).wait()

        def start_store(c, b):
            pltpu.async_copy(ob[b], out_hbm.at[pl.ds(c * C, C)], semo[b])

        def wait_store(b):
            pltpu.make_async_copy(ob[b], out_hbm.at[pl.ds(0, C)],
                                  semo[b]).wait()

        def seg_walk(base, nrows, b):
            ids_v, h_v, o_v = ib[b], hb[b], ob[b]
            SUB = 32

            def load_params(qi):
                pltpu.sync_copy(spg.at[pl.ds(qi * D, D)], gp)
                pltpu.sync_copy(spb.at[pl.ds(qi * D, D)], bp)
                gr = [gp[pl.ds(kk * 16, 16)] for kk in range(NV)]
                br = [bp[pl.ds(kk * 16, 16)] for kk in range(NV)]
                return gr, br

            for s in range(nrows // SUB):
                lo = s * SUB
                hi = lo + SUB
                q0 = extract(ids_v, lo)
                q1 = extract(ids_v, hi - 1)
                gr0, br0 = load_params(q0)

                def row0(i, carry, gr0=gr0, br0=br0):
                    for kk in range(NV):
                        sl = pl.ds(kk * 16, 16)
                        o_v[i, sl] = gr0[kk] * h_v[i, sl] + br0[kk]
                    return carry
                lax.fori_loop(lo, hi, row0, None)

                @pl.when(q1 != q0)
                def _(lo=lo, hi=hi, q0=q0, q1=q1):
                    def seg(qi, r):
                        r_end = jnp.minimum(extract(rs_v, qi + 1) - base, hi)
                        gr, br = load_params(qi)

                        def row(i, carry):
                            m = jnp.minimum(
                                jnp.minimum(i - r + 1, r_end - i),
                                1).astype(jnp.float32)
                            m = jnp.maximum(m, 0.0)
                            for kk in range(NV):
                                sl = pl.ds(kk * 16, 16)
                                new_v = gr[kk] * h_v[i, sl] + br[kk]
                                old_v = o_v[i, sl]
                                o_v[i, sl] = old_v + m * (new_v - old_v)
                            return carry
                        lax.fori_loop(lo, hi, row, None)
                        return r_end

                    r1 = jnp.minimum(extract(rs_v, q0 + 1) - base, hi)
                    lax.fori_loop(q0 + 1, q1 + 1, seg, r1)

        c0 = wid

        @pl.when(c0 < n_full)
        def _():
            start_in(c0, 0)

        @pl.when(c0 + nw < n_full)
        def _():
            start_in(c0 + nw, 1)

        def pair(t, carry):
            ca = wid + (2 * t) * nw
            cb_ = ca + nw

            @pl.when(ca < n_full)
            def _():
                wait_in(0)

                @pl.when(t >= 1)
                def _():
                    wait_store(0)
                seg_walk(ca * C, C, 0)
                start_store(ca, 0)

            @pl.when(ca + 2 * nw < n_full)
            def _():
                start_in(ca + 2 * nw, 0)

            @pl.when(cb_ < n_full)
            def _():
                wait_in(1)

                @pl.when(t >= 1)
                def _():
                    wait_store(1)
                seg_walk(cb_ * C, C, 1)
                start_store(cb_, 1)

            @pl.when(cb_ + 2 * nw < n_full)
            def _():
                start_in(cb_ + 2 * nw, 1)
            return carry

        lax.fori_loop(0, tmax, pair, None)

        ce0 = wid + (2 * (tmax - 1)) * nw

        @pl.when(ce0 < n_full)
        def _():
            wait_store(0)

        @pl.when(ce0 + nw < n_full)
        def _():
            wait_store(1)

        if tail:
            @pl.when(wid == nw - 1)
            def _():
                pltpu.sync_copy(ids_hbm.at[pl.ds(n_full * C, tail)],
                                i0.at[pl.ds(0, tail)])
                pltpu.sync_copy(h_hbm.at[pl.ds(n_full * C, tail)],
                                h0.at[pl.ds(0, tail)])
                seg_walk(n_full * C, tail, 0)
                pltpu.sync_copy(o0.at[pl.ds(0, tail)],
                                out_hbm.at[pl.ds(n_full * C, tail)])

    return k(H, ids, g2f, betaf, rs_pad)


def kernel(H, X_d, layer_idx, graph_ids, W1, b1, Wh, bh):
    skipf = (jnp.asarray(layer_idx) >= 4).astype(jnp.float32).reshape(1, 1)
    g2, beta = _make_tables(X_d, W1, b1, Wh, bh, skipf)
    ids = graph_ids.astype(jnp.int32)
    B = X_d.shape[0]
    rs = jnp.searchsorted(ids, jnp.arange(B + 1, dtype=jnp.int32),
                          side="left").astype(jnp.int32)
    rsw = ((B + 1 + 15) // 16 + 1) * 16
    rs_pad = jnp.zeros((rsw,), jnp.int32).at[:B + 1].set(rs)
    return _film_sc(H, ids, g2.reshape(-1), beta.reshape(-1), rs_pad)


# speedup vs baseline: 2.9349x; 1.0740x over previous
import functools

import jax
import jax.numpy as jnp
from jax import lax
from jax.experimental import pallas as pl
from jax.experimental.pallas import tpu as pltpu
from jax.experimental.pallas import tpu_sc as plsc


def _table_body(hd, xd_ref, w1_ref, b1_ref, wh_ref, bh_ref, skip_ref,
                g2_ref, beta_ref):
    t = lax.dot_general(xd_ref[...], w1_ref[...], (((1,), (1,)), ((), ())),
                        preferred_element_type=jnp.float32)
    t = jnp.maximum(t + b1_ref[...], 0.0)
    p = lax.dot_general(t, wh_ref[...], (((1,), (1,)), ((), ())),
                        preferred_element_type=jnp.float32)
    p = p + bh_ref[...]
    gamma = p[:, :hd]
    beta = p[:, hd:]
    g2 = 1.0 + jnp.tanh(gamma)
    s = skip_ref[0, 0]
    g2_ref[...] = g2 * (1.0 - s) + s
    beta_ref[...] = beta * (1.0 - s)


def _make_tables(X_d, W1, b1, Wh, bh, skipf):
    B, _ = X_d.shape
    hd = Wh.shape[0] // 2
    return pl.pallas_call(
        functools.partial(_table_body, hd),
        out_shape=(jax.ShapeDtypeStruct((B, hd), jnp.float32),
                   jax.ShapeDtypeStruct((B, hd), jnp.float32)),
    )(X_d, W1, b1.reshape(1, -1), Wh, bh.reshape(1, -1), skipf)


def _film_sc(H, ids, g2f, betaf, rs_pad):
    N, D = H.shape
    NV = D // 16
    C = 128
    n_full = N // C
    tail = N - n_full * C
    tabw = g2f.shape[0]
    rsw = rs_pad.shape[0]
    info = plsc.get_sparse_core_info()
    nc, ns = info.num_cores, info.num_subcores
    nw = nc * ns
    rmax = (n_full + nw - 1) // nw
    tmax = (rmax + 1) // 2
    mesh = plsc.VectorSubcoreMesh(core_axis_name="c", subcore_axis_name="s")

    @functools.partial(
        pl.kernel,
        mesh=mesh,
        out_type=jax.ShapeDtypeStruct((N, D), jnp.float32),
        scratch_types=[
            pltpu.VMEM((C + 16,), jnp.int32),
            pltpu.VMEM((C + 16,), jnp.int32),
            pltpu.VMEM((C, D), jnp.float32),
            pltpu.VMEM((C, D), jnp.float32),
            pltpu.VMEM((C, D), jnp.float32),
            pltpu.VMEM((C, D), jnp.float32),
            pltpu.VMEM((rsw,), jnp.int32),
            pltpu.VMEM((D,), jnp.float32),
            pltpu.VMEM((D,), jnp.float32),
            pltpu.VMEM_SHARED((tabw,), jnp.float32),
            pltpu.VMEM_SHARED((tabw,), jnp.float32),
            pltpu.SemaphoreType.DMA,
            pltpu.SemaphoreType.DMA,
            pltpu.SemaphoreType.DMA,
            pltpu.SemaphoreType.DMA,
        ],
    )
    def k(h_hbm, ids_hbm, g2_hbm, beta_hbm, rs_hbm, out_hbm,
          i0, i1, h0, h1, o0, o1, rs_v, gp, bp, spg, spb,
          s0, s1, so0, so1):
        wid = lax.axis_index("s") * nc + lax.axis_index("c")
        ib = (i0, i1)
        hb = (h0, h1)
        ob = (o0, o1)
        sem = (s0, s1)
        semo = (so0, so1)

        @pl.when(lax.axis_index("s") == 0)
        def _():
            pltpu.sync_copy(g2_hbm, spg)
            pltpu.sync_copy(beta_hbm, spb)
        plsc.subcore_barrier()
        pltpu.sync_copy(rs_hbm, rs_v)

        def extract(ref, idx):
            return ref[pl.ds(idx, 16)][0]

        def start_in(c, b):
            pltpu.async_copy(ids_hbm.at[pl.ds(c * C, C)],
                             ib[b].at[pl.ds(0, C)], sem[b])
            pltpu.async_copy(h_hbm.at[pl.ds(c * C, C)], hb[b], sem[b])

        def wait_in(b):
            pltpu.make_async_copy(ids_hbm.at[pl.ds(0, C)],
                                  ib[b].at[pl.ds(0, C)], sem[b]).wait()
            pltpu.make_async_copy(h_hbm.at[pl.ds(0, C)],
                                  hb[b], sem[b]).wait()

        def start_store(c, b):
            pltpu.async_copy(ob[b], out_hbm.at[pl.ds(c * C, C)], semo[b])

        def wait_store(b):
            pltpu.make_async_copy(ob[b], out_hbm.at[pl.ds(0, C)],
                                  semo[b]).wait()

        def seg_walk(base, nrows, b):
            ids_v, h_v, o_v = ib[b], hb[b], ob[b]
            SUB = 32

            def load_params(qi):
                pltpu.sync_copy(spg.at[pl.ds(qi * D, D)], gp)
                pltpu.sync_copy(spb.at[pl.ds(qi * D, D)], bp)
                gr = [gp[pl.ds(kk * 16, 16)] for kk in range(NV)]
                br = [bp[pl.ds(kk * 16, 16)] for kk in range(NV)]
                return gr, br

            prev = jnp.int32(-1)
            for s in range(nrows // SUB):
                lo = s * SUB
                hi = lo + SUB
                q0 = extract(ids_v, lo)
                q1 = extract(ids_v, hi - 1)

                @pl.when(q0 != prev)
                def _(q0=q0):
                    pltpu.sync_copy(spg.at[pl.ds(q0 * D, D)], gp)
                    pltpu.sync_copy(spb.at[pl.ds(q0 * D, D)], bp)
                prev = q1
                gr0 = [gp[pl.ds(kk * 16, 16)] for kk in range(NV)]
                br0 = [bp[pl.ds(kk * 16, 16)] for kk in range(NV)]

                def row0(i, carry, gr0=gr0, br0=br0):
                    for kk in range(NV):
                        sl = pl.ds(kk * 16, 16)
                        o_v[i, sl] = gr0[kk] * h_v[i, sl] + br0[kk]
                    return carry
                lax.fori_loop(lo, hi, row0, None)

                @pl.when(q1 != q0)
                def _(lo=lo, hi=hi, q0=q0, q1=q1):
                    def seg(qi, r):
                        r_end = jnp.minimum(extract(rs_v, qi + 1) - base, hi)
                        gr, br = load_params(qi)

                        def row(i, carry):
                            m = jnp.minimum(
                                jnp.minimum(i - r + 1, r_end - i),
                                1).astype(jnp.float32)
                            m = jnp.maximum(m, 0.0)
                            for kk in range(NV):
                                sl = pl.ds(kk * 16, 16)
                                new_v = gr[kk] * h_v[i, sl] + br[kk]
                                old_v = o_v[i, sl]
                                o_v[i, sl] = old_v + m * (new_v - old_v)
                            return carry
                        lax.fori_loop(lo, hi, row, None)
                        return r_end

                    r1 = jnp.minimum(extract(rs_v, q0 + 1) - base, hi)
                    lax.fori_loop(q0 + 1, q1 + 1, seg, r1)

        c0 = wid

        @pl.when(c0 < n_full)
        def _():
            start_in(c0, 0)

        @pl.when(c0 + nw < n_full)
        def _():
            start_in(c0 + nw, 1)

        def pair(t, carry):
            ca = wid + (2 * t) * nw
            cb_ = ca + nw

            @pl.when(ca < n_full)
            def _():
                wait_in(0)

                @pl.when(t >= 1)
                def _():
                    wait_store(0)
                seg_walk(ca * C, C, 0)
                start_store(ca, 0)

            @pl.when(ca + 2 * nw < n_full)
            def _():
                start_in(ca + 2 * nw, 0)

            @pl.when(cb_ < n_full)
            def _():
                wait_in(1)

                @pl.when(t >= 1)
                def _():
                    wait_store(1)
                seg_walk(cb_ * C, C, 1)
                start_store(cb_, 1)

            @pl.when(cb_ + 2 * nw < n_full)
            def _():
                start_in(cb_ + 2 * nw, 1)
            return carry

        lax.fori_loop(0, tmax, pair, None)

        ce0 = wid + (2 * (tmax - 1)) * nw

        @pl.when(ce0 < n_full)
        def _():
            wait_store(0)

        @pl.when(ce0 + nw < n_full)
        def _():
            wait_store(1)

        if tail:
            @pl.when(wid == nw - 1)
            def _():
                pltpu.sync_copy(ids_hbm.at[pl.ds(n_full * C, tail)],
                                i0.at[pl.ds(0, tail)])
                pltpu.sync_copy(h_hbm.at[pl.ds(n_full * C, tail)],
                                h0.at[pl.ds(0, tail)])
                seg_walk(n_full * C, tail, 0)
                pltpu.sync_copy(o0.at[pl.ds(0, tail)],
                                out_hbm.at[pl.ds(n_full * C, tail)])

    return k(H, ids, g2f, betaf, rs_pad)


def kernel(H, X_d, layer_idx, graph_ids, W1, b1, Wh, bh):
    skipf = (jnp.asarray(layer_idx) >= 4).astype(jnp.float32).reshape(1, 1)
    g2, beta = _make_tables(X_d, W1, b1, Wh, bh, skipf)
    ids = graph_ids.astype(jnp.int32)
    B = X_d.shape[0]
    rs = jnp.searchsorted(ids, jnp.arange(B + 1, dtype=jnp.int32),
                          side="left").astype(jnp.int32)
    rsw = ((B + 1 + 15) // 16 + 1) * 16
    rs_pad = jnp.zeros((rsw,), jnp.int32).at[:B + 1].set(rs)
    return _film_sc(H, ids, g2.reshape(-1), beta.reshape(-1), rs_pad)
